# baseline (speedup 1.0000x reference)
import functools

import jax
import jax.numpy as jnp
from jax import lax
from jax.experimental import pallas as pl
from jax.experimental.pallas import tpu as pltpu
from jax.experimental.pallas import tpu_sc as plsc

_info = plsc.get_sparse_core_info()
_NC, _NS = _info.num_cores, _info.num_subcores
_NW = _NC * _NS

_CHUNK = 16
_NBUF = 4
_GDEPTH = 3
_SSLOT = 3

_SC_FRAC_NUM, _SC_FRAC_DEN = 5, 8
_TROWS = 128


@functools.lru_cache(maxsize=None)
def _make_sc_gather(B, Bsc, V, D):
    assert Bsc % (_NW * _CHUNK) == 0
    b_per_w = Bsc // _NW
    n_chunks = b_per_w // _CHUNK
    mesh = plsc.VectorSubcoreMesh(core_axis_name="c", subcore_axis_name="s")

    @functools.partial(
        pl.kernel,
        out_type=jax.ShapeDtypeStruct((B, D), jnp.float32),
        mesh=mesh,
        scratch_types=[
            pltpu.VMEM((b_per_w,), jnp.int32),
            pltpu.VMEM((_NBUF, _CHUNK, D), jnp.float32),
            pltpu.VMEM_SHARED((_NS, _SSLOT, _CHUNK, D), jnp.float32),
            pltpu.SemaphoreType.DMA,
        ]
        + [pltpu.SemaphoreType.DMA] * (2 * _SSLOT),
    )
    def gather_kernel(table_hbm, idx_hbm, out_hbm, idx_v, rows_v, sp, gsem, *sems):
        csems, dsems = sems[:_SSLOT], sems[_SSLOT:]
        sid = lax.axis_index("s")
        wid = sid * _NC + lax.axis_index("c")
        base = wid * b_per_w
        pltpu.sync_copy(idx_hbm.at[pl.ds(base, b_per_w)], idx_v)

        def start_gather(g):
            return pltpu.async_copy(
                table_hbm.at[idx_v.at[pl.ds(g * _CHUNK, _CHUNK)]],
                rows_v.at[g % _NBUF],
                gsem,
            )

        def start_stage(g):
            return pltpu.async_copy(
                rows_v.at[g % _NBUF], sp.at[sid, g % _SSLOT], csems[g % _SSLOT]
            )

        def start_write(g):
            return pltpu.async_copy(
                sp.at[sid, g % _SSLOT],
                out_hbm.at[pl.ds(base + g * _CHUNK, _CHUNK)],
                dsems[g % _SSLOT],
            )

        gathers = [None] * n_chunks
        stages = [None] * n_chunks
        writes = [None] * n_chunks
        for g in range(min(_GDEPTH, n_chunks)):
            gathers[g] = start_gather(g)
        for g in range(n_chunks):
            gathers[g].wait()
            if g - _SSLOT >= 0:
                writes[g - _SSLOT].wait()
            stages[g] = start_stage(g)
            if g >= 1:
                stages[g - 1].wait()
                writes[g - 1] = start_write(g - 1)
            if g + _GDEPTH < n_chunks:
                gathers[g + _GDEPTH] = start_gather(g + _GDEPTH)
        stages[n_chunks - 1].wait()
        writes[n_chunks - 1] = start_write(n_chunks - 1)
        for g in range(max(0, n_chunks - _SSLOT), n_chunks):
            writes[g].wait()

    return gather_kernel


def _tc_gather_body(idx_sref, table_ref, o_ref, sem):
    i = pl.program_id(0)
    copies = []
    for r in range(_TROWS):
        c = pltpu.make_async_copy(
            table_ref.at[idx_sref[i * _TROWS + r]], o_ref.at[r], sem
        )
        c.start()
        copies.append(c)
    for c in copies:
        c.wait()


@functools.lru_cache(maxsize=None)
def _make_tc_gather(Btc, V, D):
    assert Btc % _TROWS == 0
    return pl.pallas_call(
        _tc_gather_body,
        grid_spec=pltpu.PrefetchScalarGridSpec(
            num_scalar_prefetch=1,
            grid=(Btc // _TROWS,),
            in_specs=[pl.BlockSpec(memory_space=pl.ANY)],
            out_specs=pl.BlockSpec((_TROWS, D), lambda i, *_: (i, 0)),
            scratch_shapes=[pltpu.SemaphoreType.DMA],
        ),
        out_shape=jax.ShapeDtypeStruct((Btc, D), jnp.float32),
    )


def kernel(tokens, W_E):
    B = tokens.size
    V, D = W_E.shape
    idx = tokens.reshape(B).astype(jnp.int32)
    Bsc = (B * _SC_FRAC_NUM // _SC_FRAC_DEN) // (_NW * _CHUNK) * (_NW * _CHUNK)
    Btc = B - Bsc
    out_sc = _make_sc_gather(B, Bsc, V, D)(W_E, idx)
    out_tc = _make_tc_gather(Btc, V, D)(idx[Bsc:], W_E)
    out = lax.dynamic_update_slice(out_sc, out_tc, (Bsc, 0))
    return out.reshape(*tokens.shape, D)
